# baseline (speedup 1.0000x reference)
import functools

import jax
import jax.numpy as jnp
from jax import lax
from jax.experimental import pallas as pl
from jax.experimental.pallas import tpu as pltpu
from jax.experimental.pallas import tpu_sc as plsc

VOCAB = 52000
HIDDEN = 768
MAX_POS = 2048
EPS = 1e-12

_INFO = plsc.get_sparse_core_info()
_NC = _INFO.num_cores
_NS = _INFO.num_subcores
_NW = _NC * _NS

_TOKENS = 4 * 2048
_TOK_PER_W = _TOKENS // _NW
_CH = 64
_NCH = _TOK_PER_W // _CH


def _sc_gather(input_ids, token_table, half, n_tokens, nch):
    mesh = plsc.VectorSubcoreMesh(core_axis_name="c", subcore_axis_name="s")
    tok_per_w = n_tokens // _NW
    B, S = input_ids.shape
    w_per_seq = S // tok_per_w
    seq0 = half * (n_tokens // S)

    @functools.partial(
        pl.kernel,
        mesh=mesh,
        out_type=jax.ShapeDtypeStruct((n_tokens, HIDDEN), jnp.float32),
        scratch_types=[
            pltpu.VMEM((tok_per_w,), jnp.int32),
            pltpu.VMEM((_CH, HIDDEN), jnp.float32),
            pltpu.VMEM((_CH, HIDDEN), jnp.float32),
            pltpu.SemaphoreType.DMA,
            pltpu.SemaphoreType.DMA,
        ],
    )
    def k(ids_ref, table_ref, out_ref, idx_v, buf0, buf1, sem0, sem1):
        wid = lax.axis_index("s") * _NC + lax.axis_index("c")
        base = wid * tok_per_w
        seq = seq0 + wid // w_per_seq
        col = (wid % w_per_seq) * tok_per_w
        pltpu.sync_copy(ids_ref.at[seq, pl.ds(col, tok_per_w)], idx_v)
        bufs = (buf0, buf1)
        sems = (sem0, sem1)
        cp = pltpu.async_copy(
            table_ref.at[idx_v.at[pl.ds(0, _CH)]], bufs[0], sems[0])
        for c in range(nch):
            cp.wait()
            if c + 1 < nch:
                cp = pltpu.async_copy(
                    table_ref.at[idx_v.at[pl.ds((c + 1) * _CH, _CH)]],
                    bufs[(c + 1) % 2], sems[(c + 1) % 2])
            pltpu.sync_copy(bufs[c % 2],
                            out_ref.at[pl.ds(base + c * _CH, _CH)])

    return k(input_ids, token_table)


def _tc_ln_body(*refs):
    g_ref, p_ref, gamma_ref, beta_ref = refs[:4]
    o_ref = refs[-1]
    x = g_ref[...] + p_ref[...]
    mean = jnp.mean(x, axis=-1, keepdims=True)
    xc = x - mean
    var = jnp.mean(xc * xc, axis=-1, keepdims=True)
    o_ref[...] = ((xc * lax.rsqrt(var + EPS)) * gamma_ref[...][None, :]
                  + beta_ref[...][None, :])


def _tc_ln_into(gathered, pos_table, gamma, beta, dst, row0, seqs):
    S = pos_table.shape[0]
    blk0 = row0 // S
    in_specs = [
        pl.BlockSpec((S, HIDDEN), lambda j: (j, 0)),
        pl.BlockSpec((S, HIDDEN), lambda j: (0, 0)),
        pl.BlockSpec((HIDDEN,), lambda j: (0,)),
        pl.BlockSpec((HIDDEN,), lambda j: (0,)),
    ]
    args = [gathered, pos_table, gamma, beta]
    aliases = {}
    if dst is not None:
        in_specs.append(pl.BlockSpec(memory_space=pltpu.MemorySpace.HBM))
        args.append(dst)
        aliases = {4: 0}
    return pl.pallas_call(
        _tc_ln_body,
        grid=(seqs,),
        in_specs=in_specs,
        out_specs=pl.BlockSpec((S, HIDDEN), lambda j: (blk0 + j, 0)),
        out_shape=jax.ShapeDtypeStruct((_TOKENS, HIDDEN), jnp.float32),
        input_output_aliases=aliases,
    )(*args)


def kernel(input_ids, token_table, pos_table, gamma, beta):
    B, S = input_ids.shape
    halves = 2
    bh = B // halves
    ntok = bh * S
    nch = ntok // _NW // _CH
    g = [_sc_gather(input_ids, token_table, h, ntok, nch)
         for h in range(halves)]
    dst = None
    for h in range(halves):
        dst = _tc_ln_into(g[h], pos_table, gamma, beta, dst, h * ntok, bh)
    return dst.reshape(B, S, HIDDEN)


# speedup vs baseline: 1.0207x; 1.0207x over previous
import functools

import jax
import jax.numpy as jnp
from jax import lax
from jax.experimental import pallas as pl
from jax.experimental.pallas import tpu as pltpu
from jax.experimental.pallas import tpu_sc as plsc

VOCAB = 52000
HIDDEN = 768
MAX_POS = 2048
EPS = 1e-12

_INFO = plsc.get_sparse_core_info()
_NC = _INFO.num_cores
_NS = _INFO.num_subcores
_NW = _NC * _NS

_TOKENS = 4 * 2048
_TOK_PER_W = _TOKENS // _NW
_CH = 64
_NCH = _TOK_PER_W // _CH


def _sc_gather(input_ids, token_table, half, n_tokens, nch):
    mesh = plsc.VectorSubcoreMesh(core_axis_name="c", subcore_axis_name="s")
    tok_per_w = n_tokens // _NW
    B, S = input_ids.shape
    w_per_seq = S // tok_per_w
    seq0 = half * (n_tokens // S)

    @functools.partial(
        pl.kernel,
        mesh=mesh,
        out_type=jax.ShapeDtypeStruct((n_tokens, HIDDEN), jnp.float32),
        scratch_types=[
            pltpu.VMEM((tok_per_w,), jnp.int32),
            pltpu.VMEM((_CH, HIDDEN), jnp.float32),
            pltpu.VMEM((_CH, HIDDEN), jnp.float32),
            pltpu.SemaphoreType.DMA,
            pltpu.SemaphoreType.DMA,
            pltpu.SemaphoreType.DMA,
            pltpu.SemaphoreType.DMA,
        ],
    )
    def k(ids_ref, table_ref, out_ref, idx_v, buf0, buf1,
          sem0, sem1, wsem0, wsem1):
        wsems = (wsem0, wsem1)
        wid = lax.axis_index("s") * _NC + lax.axis_index("c")
        base = wid * tok_per_w
        seq = seq0 + wid // w_per_seq
        col = (wid % w_per_seq) * tok_per_w
        pltpu.sync_copy(ids_ref.at[seq, pl.ds(col, tok_per_w)], idx_v)
        bufs = (buf0, buf1)
        gsems = (sem0, sem1)
        gcps = [pltpu.async_copy(
                    table_ref.at[idx_v.at[pl.ds(c * _CH, _CH)]],
                    bufs[c], gsems[c])
                for c in range(nch)]
        wcps = []
        for c in range(nch):
            gcps[c].wait()
            wcps.append(pltpu.async_copy(
                bufs[c], out_ref.at[pl.ds(base + c * _CH, _CH)], wsems[c]))
        for w in wcps:
            w.wait()

    return k(input_ids, token_table)


def _tc_ln_body(*refs):
    g_ref, p_ref, gamma_ref, beta_ref = refs[:4]
    o_ref = refs[-1]
    x = g_ref[...] + p_ref[...]
    mean = jnp.mean(x, axis=-1, keepdims=True)
    xc = x - mean
    var = jnp.mean(xc * xc, axis=-1, keepdims=True)
    o_ref[...] = ((xc * lax.rsqrt(var + EPS)) * gamma_ref[...][None, :]
                  + beta_ref[...][None, :])


def _tc_ln_into(gathered, pos_table, gamma, beta, dst, row0, seqs):
    S = pos_table.shape[0]
    blk0 = row0 // S
    in_specs = [
        pl.BlockSpec((S, HIDDEN), lambda j: (j, 0)),
        pl.BlockSpec((S, HIDDEN), lambda j: (0, 0)),
        pl.BlockSpec((HIDDEN,), lambda j: (0,)),
        pl.BlockSpec((HIDDEN,), lambda j: (0,)),
    ]
    args = [gathered, pos_table, gamma, beta]
    aliases = {}
    if dst is not None:
        in_specs.append(pl.BlockSpec(memory_space=pltpu.MemorySpace.HBM))
        args.append(dst)
        aliases = {4: 0}
    return pl.pallas_call(
        _tc_ln_body,
        grid=(seqs,),
        in_specs=in_specs,
        out_specs=pl.BlockSpec((S, HIDDEN), lambda j: (blk0 + j, 0)),
        out_shape=jax.ShapeDtypeStruct((_TOKENS, HIDDEN), jnp.float32),
        input_output_aliases=aliases,
    )(*args)


def kernel(input_ids, token_table, pos_table, gamma, beta):
    B, S = input_ids.shape
    halves = 2
    bh = B // halves
    ntok = bh * S
    nch = ntok // _NW // _CH
    g = [_sc_gather(input_ids, token_table, h, ntok, nch)
         for h in range(halves)]
    dst = None
    for h in range(halves):
        dst = _tc_ln_into(g[h], pos_table, gamma, beta, dst, h * ntok, bh)
    return dst.reshape(B, S, HIDDEN)
